# Initial kernel scaffold; baseline (speedup 1.0000x reference)
import jax
import jax.numpy as jnp
from jax.experimental import pallas as pl


def kernel(x, edge_index, edge_attr, ln_g, ln_b, Wq, bq, Wk, bk, Wv, bv, e0w1, e0b1, e0w2, e0b2, e1w1, e1b1, e1w2, e1b2):
    raise NotImplementedError("write your pallas kernel here")


# speedup vs baseline: 10.0973x; 10.0973x over previous
import jax
import jax.numpy as jnp
from jax import lax
from jax.experimental import pallas as pl
from jax.experimental.pallas import tpu as pltpu
from jax.experimental.pallas import tpu_sc as plsc

N = 10000
E = 320000
D = 128
H = 8
C = 16

NUM_CORES = 2
NUM_SUBCORES = 16
NW = NUM_CORES * NUM_SUBCORES
E_PER_W = E // NW
CHUNK = 80
N_CHUNKS = E_PER_W // CHUNK


def _ln_qkv_body(x_ref, g_ref, b_ref, wq_ref, bq_ref, wk_ref, bk_ref,
                 wv_ref, bv_ref, q_ref, k_ref, v_ref):
  x = x_ref[...]
  mu = jnp.mean(x, axis=1, keepdims=True)
  var = jnp.mean((x - mu) * (x - mu), axis=1, keepdims=True)
  xn = (x - mu) * lax.rsqrt(var + 1e-5) * g_ref[...] + b_ref[...]
  dn = (((1,), (1,)), ((), ()))
  q_ref[...] = lax.dot_general(xn, wq_ref[...], dn,
                               preferred_element_type=jnp.float32) + bq_ref[...]
  k_ref[...] = lax.dot_general(xn, wk_ref[...], dn,
                               preferred_element_type=jnp.float32) + bk_ref[...]
  v_ref[...] = lax.dot_general(xn, wv_ref[...], dn,
                               preferred_element_type=jnp.float32) + bv_ref[...]


def _ln_qkv(x, ln_g, ln_b, Wq, bq, Wk, bk, Wv, bv):
  blk = 1000
  grid = N // blk
  full = lambda shape: pl.BlockSpec(shape, lambda i: (0,) * len(shape))
  rows = pl.BlockSpec((blk, D), lambda i: (i, 0))
  return pl.pallas_call(
      _ln_qkv_body,
      grid=(grid,),
      in_specs=[rows, full((D,)), full((D,)), full((D, D)), full((D,)),
                full((D, D)), full((D,)), full((D, D)), full((D,))],
      out_specs=[rows, rows, rows],
      out_shape=[jax.ShapeDtypeStruct((N, D), jnp.float32)] * 3,
  )(x, ln_g.reshape(D), ln_b.reshape(D), Wq, bq, Wk, bk, Wv, bv)


def _edge_mlp_body(ea_ref, w01_ref, b01_ref, w02_ref, b02_ref,
                   w11_ref, b11_ref, w12_ref, b12_ref, mod_ref):
  ea = ea_ref[...]
  ef = ea[:, :14]
  dn = (((1,), (1,)), ((), ()))
  h0 = jax.nn.relu(lax.dot_general(ef, w01_ref[...], dn,
                                   preferred_element_type=jnp.float32)
                   + b01_ref[...])
  h0 = lax.dot_general(h0, w02_ref[...], dn,
                       preferred_element_type=jnp.float32) + b02_ref[...]
  h1 = jax.nn.relu(lax.dot_general(ef, w11_ref[...], dn,
                                   preferred_element_type=jnp.float32)
                   + b11_ref[...])
  h1 = lax.dot_general(h1, w12_ref[...], dn,
                       preferred_element_type=jnp.float32) + b12_ref[...]
  is0 = (ea[:, 14] >= ea[:, 15])[:, None]
  mod_ref[...] = jnp.where(is0, h0, h1)


def _edge_mlp(edge_attr, e0w1, e0b1, e0w2, e0b2, e1w1, e1b1, e1w2, e1b2):
  blk = 4000
  grid = E // blk
  full = lambda shape: pl.BlockSpec(shape, lambda i: (0,) * len(shape))
  return pl.pallas_call(
      _edge_mlp_body,
      grid=(grid,),
      in_specs=[pl.BlockSpec((blk, 16), lambda i: (i, 0)),
                full((32, 14)), full((32,)), full((H, 32)), full((H,)),
                full((32, 14)), full((32,)), full((H, 32)), full((H,))],
      out_specs=pl.BlockSpec((blk, H), lambda i: (i, 0)),
      out_shape=jax.ShapeDtypeStruct((E, H), jnp.float32),
  )(edge_attr, e0w1, e0b1, e0w2, e0b2, e1w1, e1b1, e1w2, e1b2)


def _sca_body(q_hbm, k_hbm, v_hbm, src_hbm, dst_hbm, mod_hbm, zn_hbm,
              onum_hbm, ow_hbm,
              num_sh, sidx, didx, q_buf, k_buf, mod_buf, w_buf, sem):
  cid = lax.axis_index("c")
  sid = lax.axis_index("s")
  wid = cid * NUM_SUBCORES + sid

  @pl.when(sid == 0)
  def _init():
    pltpu.sync_copy(zn_hbm, num_sh)
  plsc.subcore_barrier()

  lanes = lax.iota(jnp.int32, 16)
  e_base = wid * E_PER_W

  def chunk_body(ch):
    base = e_base + ch * CHUNK
    pltpu.sync_copy(src_hbm.at[pl.ds(base, CHUNK)], sidx)
    pltpu.sync_copy(dst_hbm.at[pl.ds(base, CHUNK)], didx)
    pltpu.sync_copy(mod_hbm.at[pl.ds(base, CHUNK)], mod_buf)
    cq = pltpu.async_copy(q_hbm.at[didx], q_buf, sem)
    ck = pltpu.async_copy(k_hbm.at[sidx], k_buf, sem)
    cq.wait()
    ck.wait()

    def logits_group(g, _):
      eidx = lanes + g * 16

      def h_body(h, _):
        acc = jnp.zeros((16,), jnp.float32)
        for c in range(C):
          colv = jnp.broadcast_to(h * C + c, (16,)).astype(jnp.int32)
          qv = plsc.load_gather(q_buf, [eidx, colv])
          kv = plsc.load_gather(k_buf, [eidx, colv])
          acc = acc + qv * kv
        hv = jnp.broadcast_to(h, (16,)).astype(jnp.int32)
        modv = plsc.load_gather(mod_buf, [eidx, hv])
        w = jnp.exp(acc * 0.25 + modv)
        plsc.store_scatter(w_buf, [eidx, hv], w)
        return 0

      lax.fori_loop(0, H, h_body, 0)
      return 0

    lax.fori_loop(0, CHUNK // 16, logits_group, 0)

    pltpu.async_copy(v_hbm.at[sidx], q_buf, sem).wait()

    def msg_group(g, _):
      eidx = lanes + g * 16

      def h_body(h, _):
        hv = jnp.broadcast_to(h, (16,)).astype(jnp.int32)
        w = plsc.load_gather(w_buf, [eidx, hv])
        for c in range(C):
          colv = jnp.broadcast_to(h * C + c, (16,)).astype(jnp.int32)
          vv = plsc.load_gather(q_buf, [eidx, colv])
          plsc.store_scatter(q_buf, [eidx, colv], vv * w)
        return 0

      lax.fori_loop(0, H, h_body, 0)
      return 0

    lax.fori_loop(0, CHUNK // 16, msg_group, 0)

    pltpu.sync_copy(q_buf, num_sh.at[didx], add=True)
    pltpu.sync_copy(w_buf, ow_hbm.at[pl.ds(base, CHUNK)])

  pl.loop(0, N_CHUNKS)(chunk_body)

  plsc.subcore_barrier()

  @pl.when(sid == 0)
  def _writeout():
    pltpu.sync_copy(num_sh, onum_hbm.at[pl.ds(cid * N, N)])


def _sc_attention(q, k, v, src, dst, mod):
  zn = jnp.zeros((N, D), jnp.float32)
  mesh = plsc.VectorSubcoreMesh(core_axis_name="c", subcore_axis_name="s")
  fn = pl.kernel(
      _sca_body,
      out_type=(jax.ShapeDtypeStruct((2 * N, D), jnp.float32),
                jax.ShapeDtypeStruct((E, H), jnp.float32)),
      mesh=mesh,
      scratch_types=(
          pltpu.VMEM_SHARED((N, D), jnp.float32),
          pltpu.VMEM((CHUNK,), jnp.int32),
          pltpu.VMEM((CHUNK,), jnp.int32),
          pltpu.VMEM((CHUNK, D), jnp.float32),
          pltpu.VMEM((CHUNK, D), jnp.float32),
          pltpu.VMEM((CHUNK, H), jnp.float32),
          pltpu.VMEM((CHUNK, H), jnp.float32),
          pltpu.SemaphoreType.DMA,
      ),
      compiler_params=pltpu.CompilerParams(needs_layout_passes=False,
                                           use_tc_tiling_on_sc=True),
  )
  return fn(q, k, v, src, dst, mod, zn)


def _scb_body(w_hbm, dst_hbm, oden_hbm, den_acc, didx, w_buf, sem):
  del sem
  cid = lax.axis_index("c")
  sid = lax.axis_index("s")
  wid = cid * NUM_SUBCORES + sid

  zv = jnp.zeros((16,), jnp.float32)

  def zbody(i, _):
    den_acc[pl.ds(i * 16, 16)] = zv
    return 0

  lax.fori_loop(0, N * H // 16, zbody, 0, unroll=8)

  lanes = lax.iota(jnp.int32, 16)
  e_base = wid * E_PER_W

  def chunk_body(ch):
    base = e_base + ch * CHUNK
    pltpu.sync_copy(dst_hbm.at[pl.ds(base, CHUNK)], didx)
    pltpu.sync_copy(w_hbm.at[pl.ds(base, CHUNK)], w_buf)

    def group(g, _):
      eidx = lanes + g * 16
      dstv = plsc.load_gather(didx, [eidx])

      def h_body(h, _):
        hv = jnp.broadcast_to(h, (16,)).astype(jnp.int32)
        wv = plsc.load_gather(w_buf, [eidx, hv])
        plsc.addupdate_scatter(den_acc, [dstv * H + hv], wv)
        return 0

      lax.fori_loop(0, H, h_body, 0)
      return 0

    lax.fori_loop(0, CHUNK // 16, group, 0)

  pl.loop(0, N_CHUNKS)(chunk_body)

  pltpu.sync_copy(den_acc, oden_hbm.at[pl.ds(wid * (N * H), N * H)])


def _sc_den(w, dst):
  mesh = plsc.VectorSubcoreMesh(core_axis_name="c", subcore_axis_name="s")
  fn = pl.kernel(
      _scb_body,
      out_type=jax.ShapeDtypeStruct((NW * N * H,), jnp.float32),
      mesh=mesh,
      scratch_types=(
          pltpu.VMEM((N * H,), jnp.float32),
          pltpu.VMEM((CHUNK,), jnp.int32),
          pltpu.VMEM((CHUNK, H), jnp.float32),
          pltpu.SemaphoreType.DMA,
      ),
      compiler_params=pltpu.CompilerParams(needs_layout_passes=False),
  )
  return fn(w, dst)


def _combine_body(n0_ref, n1_ref, d_ref, x_ref, o_ref):
  n = n0_ref[...] + n1_ref[...]
  d8 = jnp.sum(d_ref[...], axis=0)
  r = lax.broadcasted_iota(jnp.int32, (H, D), 0)
  c = lax.broadcasted_iota(jnp.int32, (H, D), 1)
  expand = (c // C == r).astype(jnp.float32)
  dn = (((1,), (0,)), ((), ()))
  dexp = lax.dot_general(d8, expand, dn, preferred_element_type=jnp.float32)
  y = n / (dexp + 1e-12)
  g = 0.5 * y * (1.0 + lax.erf(y * 0.7071067811865475))
  o_ref[...] = g + x_ref[...]


def _combine(num, den_parts, x):
  blk = 1000
  grid = N // blk
  return pl.pallas_call(
      _combine_body,
      grid=(grid,),
      in_specs=[pl.BlockSpec((blk, D), lambda i: (i, 0)),
                pl.BlockSpec((blk, D), lambda i: (i + grid, 0)),
                pl.BlockSpec((NW, blk, H), lambda i: (0, i, 0)),
                pl.BlockSpec((blk, D), lambda i: (i, 0))],
      out_specs=pl.BlockSpec((blk, D), lambda i: (i, 0)),
      out_shape=jax.ShapeDtypeStruct((N, D), jnp.float32),
  )(num, num, den_parts, x)


@jax.jit
def kernel(x, edge_index, edge_attr, ln_g, ln_b, Wq, bq, Wk, bk, Wv, bv,
           e0w1, e0b1, e0w2, e0b2, e1w1, e1b1, e1w2, e1b2):
  q, k, v = _ln_qkv(x, ln_g, ln_b, Wq, bq, Wk, bk, Wv, bv)
  mod = _edge_mlp(edge_attr, e0w1, e0b1, e0w2, e0b2, e1w1, e1b1, e1w2, e1b2)
  src = edge_index[0]
  dst = edge_index[1]
  num, w = _sc_attention(q, k, v, src, dst, mod)
  den_flat = _sc_den(w, dst)
  den_parts = den_flat.reshape(NW, N, H)
  return _combine(num, den_parts, x)


# speedup vs baseline: 10.2893x; 1.0190x over previous
import jax
import jax.numpy as jnp
from jax import lax
from jax.experimental import pallas as pl
from jax.experimental.pallas import tpu as pltpu
from jax.experimental.pallas import tpu_sc as plsc

N = 10000
E = 320000
D = 128
H = 8
C = 16

NUM_CORES = 2
NUM_SUBCORES = 16
NW = NUM_CORES * NUM_SUBCORES
E_PER_W = E // NW
CHUNK = 80
N_CHUNKS = E_PER_W // CHUNK


def _ln_qkv_body(x_ref, g_ref, b_ref, wq_ref, bq_ref, wk_ref, bk_ref,
                 wv_ref, bv_ref, q_ref, k_ref, v_ref):
  x = x_ref[...]
  mu = jnp.mean(x, axis=1, keepdims=True)
  var = jnp.mean((x - mu) * (x - mu), axis=1, keepdims=True)
  xn = (x - mu) * lax.rsqrt(var + 1e-5) * g_ref[...] + b_ref[...]
  dn = (((1,), (1,)), ((), ()))
  q_ref[...] = lax.dot_general(xn, wq_ref[...], dn,
                               preferred_element_type=jnp.float32) + bq_ref[...]
  k_ref[...] = lax.dot_general(xn, wk_ref[...], dn,
                               preferred_element_type=jnp.float32) + bk_ref[...]
  v_ref[...] = lax.dot_general(xn, wv_ref[...], dn,
                               preferred_element_type=jnp.float32) + bv_ref[...]


def _ln_qkv(x, ln_g, ln_b, Wq, bq, Wk, bk, Wv, bv):
  blk = 1000
  grid = N // blk
  full = lambda shape: pl.BlockSpec(shape, lambda i: (0,) * len(shape))
  rows = pl.BlockSpec((blk, D), lambda i: (i, 0))
  return pl.pallas_call(
      _ln_qkv_body,
      grid=(grid,),
      in_specs=[rows, full((D,)), full((D,)), full((D, D)), full((D,)),
                full((D, D)), full((D,)), full((D, D)), full((D,))],
      out_specs=[rows, rows, rows],
      out_shape=[jax.ShapeDtypeStruct((N, D), jnp.float32)] * 3,
  )(x, ln_g.reshape(D), ln_b.reshape(D), Wq, bq, Wk, bk, Wv, bv)


def _edge_mlp_body(ea_ref, w01_ref, b01_ref, w02_ref, b02_ref,
                   w11_ref, b11_ref, w12_ref, b12_ref, mod_ref):
  ea = ea_ref[...]
  ef = ea[:, :14]
  dn = (((1,), (1,)), ((), ()))
  h0 = jax.nn.relu(lax.dot_general(ef, w01_ref[...], dn,
                                   preferred_element_type=jnp.float32)
                   + b01_ref[...])
  h0 = lax.dot_general(h0, w02_ref[...], dn,
                       preferred_element_type=jnp.float32) + b02_ref[...]
  h1 = jax.nn.relu(lax.dot_general(ef, w11_ref[...], dn,
                                   preferred_element_type=jnp.float32)
                   + b11_ref[...])
  h1 = lax.dot_general(h1, w12_ref[...], dn,
                       preferred_element_type=jnp.float32) + b12_ref[...]
  is0 = (ea[:, 14] >= ea[:, 15])[:, None]
  mod_ref[...] = jnp.where(is0, h0, h1)


def _edge_mlp(edge_attr, e0w1, e0b1, e0w2, e0b2, e1w1, e1b1, e1w2, e1b2):
  blk = 4000
  grid = E // blk
  full = lambda shape: pl.BlockSpec(shape, lambda i: (0,) * len(shape))
  return pl.pallas_call(
      _edge_mlp_body,
      grid=(grid,),
      in_specs=[pl.BlockSpec((blk, 16), lambda i: (i, 0)),
                full((32, 14)), full((32,)), full((H, 32)), full((H,)),
                full((32, 14)), full((32,)), full((H, 32)), full((H,))],
      out_specs=pl.BlockSpec((blk, H), lambda i: (i, 0)),
      out_shape=jax.ShapeDtypeStruct((E, H), jnp.float32),
  )(edge_attr, e0w1, e0b1, e0w2, e0b2, e1w1, e1b1, e1w2, e1b2)


def _sca_body(q_hbm, k_hbm, v_hbm, src_hbm, dst_hbm, mod_hbm, zn_hbm,
              onum_hbm, ow_hbm,
              num_sh, sidx, didx, q_buf, k_buf, v_buf, mod_buf, sem):
  cid = lax.axis_index("c")
  sid = lax.axis_index("s")
  wid = cid * NUM_SUBCORES + sid

  @pl.when(sid == 0)
  def _init():
    pltpu.sync_copy(zn_hbm, num_sh)
  plsc.subcore_barrier()

  lanes = lax.iota(jnp.int32, 16)
  e_base = wid * E_PER_W

  def chunk_body(ch):
    base = e_base + ch * CHUNK
    pltpu.sync_copy(src_hbm.at[pl.ds(base, CHUNK)], sidx)
    pltpu.sync_copy(dst_hbm.at[pl.ds(base, CHUNK)], didx)
    pltpu.sync_copy(mod_hbm.at[pl.ds(base, CHUNK)], mod_buf)
    cq = pltpu.async_copy(q_hbm.at[didx], q_buf, sem)
    ck = pltpu.async_copy(k_hbm.at[sidx], k_buf, sem)
    cv = pltpu.async_copy(v_hbm.at[sidx], v_buf, sem)
    cq.wait()
    ck.wait()
    cv.wait()

    def logits_group(g, _):
      eidx = lanes + g * 16

      def h_body(h, _):
        acc = jnp.zeros((16,), jnp.float32)
        for c in range(C):
          colv = jnp.broadcast_to(h * C + c, (16,)).astype(jnp.int32)
          qv = plsc.load_gather(q_buf, [eidx, colv])
          kv = plsc.load_gather(k_buf, [eidx, colv])
          acc = acc + qv * kv
        hv = jnp.broadcast_to(h, (16,)).astype(jnp.int32)
        modv = plsc.load_gather(mod_buf, [eidx, hv])
        w = jnp.exp(acc * 0.25 + modv)
        plsc.store_scatter(mod_buf, [eidx, hv], w)
        return 0

      lax.fori_loop(0, H, h_body, 0)
      return 0

    lax.fori_loop(0, CHUNK // 16, logits_group, 0)

    def msg_group(g, _):
      eidx = lanes + g * 16

      def h_body(h, _):
        hv = jnp.broadcast_to(h, (16,)).astype(jnp.int32)
        w = plsc.load_gather(mod_buf, [eidx, hv])
        for c in range(C):
          colv = jnp.broadcast_to(h * C + c, (16,)).astype(jnp.int32)
          vv = plsc.load_gather(v_buf, [eidx, colv])
          plsc.store_scatter(v_buf, [eidx, colv], vv * w)
        return 0

      lax.fori_loop(0, H, h_body, 0)
      return 0

    lax.fori_loop(0, CHUNK // 16, msg_group, 0)

    pltpu.sync_copy(v_buf, num_sh.at[didx], add=True)
    pltpu.sync_copy(mod_buf, ow_hbm.at[pl.ds(base, CHUNK)])

  pl.loop(0, N_CHUNKS)(chunk_body)

  plsc.subcore_barrier()

  @pl.when(sid == 0)
  def _writeout():
    pltpu.sync_copy(num_sh, onum_hbm.at[pl.ds(cid * N, N)])


def _sc_attention(q, k, v, src, dst, mod):
  zn = jnp.zeros((N, D), jnp.float32)
  mesh = plsc.VectorSubcoreMesh(core_axis_name="c", subcore_axis_name="s")
  fn = pl.kernel(
      _sca_body,
      out_type=(jax.ShapeDtypeStruct((2 * N, D), jnp.float32),
                jax.ShapeDtypeStruct((E, H), jnp.float32)),
      mesh=mesh,
      scratch_types=(
          pltpu.VMEM_SHARED((N, D), jnp.float32),
          pltpu.VMEM((CHUNK,), jnp.int32),
          pltpu.VMEM((CHUNK,), jnp.int32),
          pltpu.VMEM((CHUNK, D), jnp.float32),
          pltpu.VMEM((CHUNK, D), jnp.float32),
          pltpu.VMEM((CHUNK, D), jnp.float32),
          pltpu.VMEM((CHUNK, H), jnp.float32),
          pltpu.SemaphoreType.DMA,
      ),
      compiler_params=pltpu.CompilerParams(needs_layout_passes=False,
                                           use_tc_tiling_on_sc=True),
  )
  return fn(q, k, v, src, dst, mod, zn)


def _scb_body(w_hbm, dst_hbm, oden_hbm, den_acc, didx, w_buf, sem):
  del sem
  cid = lax.axis_index("c")
  sid = lax.axis_index("s")
  wid = cid * NUM_SUBCORES + sid

  zv = jnp.zeros((16,), jnp.float32)

  def zbody(i, _):
    den_acc[pl.ds(i * 16, 16)] = zv
    return 0

  lax.fori_loop(0, N * H // 16, zbody, 0, unroll=8)

  lanes = lax.iota(jnp.int32, 16)
  e_base = wid * E_PER_W

  def chunk_body(ch):
    base = e_base + ch * CHUNK
    pltpu.sync_copy(dst_hbm.at[pl.ds(base, CHUNK)], didx)
    pltpu.sync_copy(w_hbm.at[pl.ds(base, CHUNK)], w_buf)

    def group(g, _):
      eidx = lanes + g * 16
      dstv = plsc.load_gather(didx, [eidx])

      def h_body(h, _):
        hv = jnp.broadcast_to(h, (16,)).astype(jnp.int32)
        wv = plsc.load_gather(w_buf, [eidx, hv])
        plsc.addupdate_scatter(den_acc, [dstv * H + hv], wv)
        return 0

      lax.fori_loop(0, H, h_body, 0)
      return 0

    lax.fori_loop(0, CHUNK // 16, group, 0)

  pl.loop(0, N_CHUNKS)(chunk_body)

  pltpu.sync_copy(den_acc, oden_hbm.at[pl.ds(wid * (N * H), N * H)])


def _sc_den(w, dst):
  mesh = plsc.VectorSubcoreMesh(core_axis_name="c", subcore_axis_name="s")
  fn = pl.kernel(
      _scb_body,
      out_type=jax.ShapeDtypeStruct((NW * N * H,), jnp.float32),
      mesh=mesh,
      scratch_types=(
          pltpu.VMEM((N * H,), jnp.float32),
          pltpu.VMEM((CHUNK,), jnp.int32),
          pltpu.VMEM((CHUNK, H), jnp.float32),
          pltpu.SemaphoreType.DMA,
      ),
      compiler_params=pltpu.CompilerParams(needs_layout_passes=False),
  )
  return fn(w, dst)


def _combine_body(n0_ref, n1_ref, d_ref, x_ref, o_ref):
  n = n0_ref[...] + n1_ref[...]
  d8 = jnp.sum(d_ref[...], axis=0)
  r = lax.broadcasted_iota(jnp.int32, (H, D), 0)
  c = lax.broadcasted_iota(jnp.int32, (H, D), 1)
  expand = (c // C == r).astype(jnp.float32)
  dn = (((1,), (0,)), ((), ()))
  dexp = lax.dot_general(d8, expand, dn, preferred_element_type=jnp.float32)
  y = n / (dexp + 1e-12)
  g = 0.5 * y * (1.0 + lax.erf(y * 0.7071067811865475))
  o_ref[...] = g + x_ref[...]


def _combine(num, den_parts, x):
  blk = 1000
  grid = N // blk
  return pl.pallas_call(
      _combine_body,
      grid=(grid,),
      in_specs=[pl.BlockSpec((blk, D), lambda i: (i, 0)),
                pl.BlockSpec((blk, D), lambda i: (i + grid, 0)),
                pl.BlockSpec((NW, blk, H), lambda i: (0, i, 0)),
                pl.BlockSpec((blk, D), lambda i: (i, 0))],
      out_specs=pl.BlockSpec((blk, D), lambda i: (i, 0)),
      out_shape=jax.ShapeDtypeStruct((N, D), jnp.float32),
  )(num, num, den_parts, x)


@jax.jit
def kernel(x, edge_index, edge_attr, ln_g, ln_b, Wq, bq, Wk, bk, Wv, bv,
           e0w1, e0b1, e0w2, e0b2, e1w1, e1b1, e1w2, e1b2):
  q, k, v = _ln_qkv(x, ln_g, ln_b, Wq, bq, Wk, bk, Wv, bv)
  mod = _edge_mlp(edge_attr, e0w1, e0b1, e0w2, e0b2, e1w1, e1b1, e1w2, e1b2)
  src = edge_index[0]
  dst = edge_index[1]
  num, w = _sc_attention(q, k, v, src, dst, mod)
  den_flat = _sc_den(w, dst)
  den_parts = den_flat.reshape(NW, N, H)
  return _combine(num, den_parts, x)
